# Initial kernel scaffold; baseline (speedup 1.0000x reference)
import jax
import jax.numpy as jnp
from jax.experimental import pallas as pl


def kernel(edge_index, user_emb, movie_emb, W1, b1, W2, b2):
    raise NotImplementedError("write your pallas kernel here")


# speedup vs baseline: 4.7692x; 4.7692x over previous
import functools

import jax
import jax.numpy as jnp
from jax import lax
from jax.experimental import pallas as pl
from jax.experimental.pallas import tpu as pltpu
from jax.experimental.pallas import tpu_sc as plsc

N_NODES = 10000
NP = 10240
TRASH = 10000
E = 160000
BATCH = 128
NC, NS = 2, 16
NB = 80
EPT = NB * BATCH
EP = EPT * NS
RPT = NP // NS

_MESH = plsc.VectorSubcoreMesh(
    core_axis_name="c", subcore_axis_name="s", num_cores=NC, num_subcores=NS
)


def _deg_body(dst2d_hbm, ones_hbm, zeros_hbm, out_hbm, dst_v, ones_v, acc_sh):
    s = lax.axis_index("s")
    c = lax.axis_index("c")
    pltpu.sync_copy(dst2d_hbm.at[pl.ds(s * NB, NB)], dst_v)
    pltpu.sync_copy(ones_hbm, ones_v)
    pltpu.sync_copy(zeros_hbm, acc_sh.at[pl.ds(s * RPT, RPT)])
    plsc.subcore_barrier()

    @pl.loop(0, NB)
    def _(j):
        pltpu.sync_copy(ones_v, acc_sh.at[dst_v.at[j]], add=True)

    plsc.subcore_barrier()
    pltpu.sync_copy(
        acc_sh.at[pl.ds(s * RPT, RPT)], out_hbm.at[c].at[pl.ds(s * RPT, RPT)]
    )


_deg_call = functools.partial(
    pl.kernel,
    out_type=jax.ShapeDtypeStruct((NC, NP, 128), jnp.float32),
    mesh=_MESH,
    scratch_types=[
        pltpu.VMEM((NB, BATCH), jnp.int32),
        pltpu.VMEM((BATCH, 128), jnp.float32),
        pltpu.VMEM_SHARED((NP, 128), jnp.float32),
    ],
)(_deg_body)


def _make_agg(C):
    CPC = C // NC

    def body(src_hbm, dst2d_hbm, h_hbm, zeros_hbm, out_hbm,
             src_v, dst_v, rows_v, acc_sh):
        s = lax.axis_index("s")
        c = lax.axis_index("c")
        pltpu.sync_copy(src_hbm.at[pl.ds(s * EPT, EPT)], src_v)
        pltpu.sync_copy(dst2d_hbm.at[pl.ds(s * NB, NB)], dst_v)
        for cc in range(CPC):
            ch = c * CPC + cc
            pltpu.sync_copy(zeros_hbm, acc_sh.at[pl.ds(s * RPT, RPT)])
            plsc.subcore_barrier()
            table = h_hbm.at[ch]

            @pl.loop(0, NB)
            def _(j):
                idx = src_v.at[pl.ds(j * BATCH, BATCH)]
                pltpu.sync_copy(table.at[idx], rows_v)
                pltpu.sync_copy(rows_v, acc_sh.at[dst_v.at[j]], add=True)

            plsc.subcore_barrier()
            pltpu.sync_copy(
                acc_sh.at[pl.ds(s * RPT, RPT)],
                out_hbm.at[ch].at[pl.ds(s * RPT, RPT)],
            )

    return pl.kernel(
        body,
        out_type=jax.ShapeDtypeStruct((C, NP, 128), jnp.float32),
        mesh=_MESH,
        scratch_types=[
            pltpu.VMEM((EPT,), jnp.int32),
            pltpu.VMEM((NB, BATCH), jnp.int32),
            pltpu.VMEM((BATCH, 128), jnp.float32),
            pltpu.VMEM_SHARED((NP, 128), jnp.float32),
        ],
    )


_agg4 = _make_agg(4)
_agg2 = _make_agg(2)


def _mm_body(x_ref, w_ref, deg_ref, o_ref):
    p = jnp.dot(x_ref[...], w_ref[...], preferred_element_type=jnp.float32)
    o_ref[0] = p * lax.rsqrt(deg_ref[...] + 1.0)


def _mm(x, w, deg, c_out, bm=256):
    d_in = x.shape[1]
    return pl.pallas_call(
        _mm_body,
        grid=(NP // bm, c_out),
        in_specs=[
            pl.BlockSpec((bm, d_in), lambda i, j: (i, 0)),
            pl.BlockSpec((d_in, 128), lambda i, j: (0, j)),
            pl.BlockSpec((bm, 128), lambda i, j: (i, 0)),
        ],
        out_specs=pl.BlockSpec((1, bm, 128), lambda i, j: (j, i, 0)),
        out_shape=jax.ShapeDtypeStruct((c_out, NP, 128), jnp.float32),
        compiler_params=pltpu.CompilerParams(
            dimension_semantics=("parallel", "parallel")
        ),
    )(x, w, deg)


def _combine_body(a_ref, h_ref, deg_ref, b_ref, o_ref, *, relu):
    dinv = lax.rsqrt(deg_ref[...] + 1.0)
    v = dinv * (a_ref[0] + h_ref[0]) + b_ref[0]
    if relu:
        v = jnp.maximum(v, 0.0)
    o_ref[...] = v


def _combine(agg, h, deg, b2d, relu, bm=256):
    c = agg.shape[0]
    return pl.pallas_call(
        functools.partial(_combine_body, relu=relu),
        grid=(NP // bm, c),
        in_specs=[
            pl.BlockSpec((1, bm, 128), lambda i, j: (j, i, 0)),
            pl.BlockSpec((1, bm, 128), lambda i, j: (j, i, 0)),
            pl.BlockSpec((bm, 128), lambda i, j: (i, 0)),
            pl.BlockSpec((1, 1, 128), lambda i, j: (j, 0, 0)),
        ],
        out_specs=pl.BlockSpec((bm, 128), lambda i, j: (i, j)),
        out_shape=jax.ShapeDtypeStruct((NP, c * 128), jnp.float32),
        compiler_params=pltpu.CompilerParams(
            dimension_semantics=("parallel", "parallel")
        ),
    )(agg, h, deg, b2d)


def kernel(edge_index, user_emb, movie_emb, W1, b1, W2, b2):
    src = edge_index[0]
    dst = edge_index[1]
    pad = EP - E
    src_p = jnp.concatenate([src, jnp.zeros((pad,), src.dtype)])
    dst_p = jnp.concatenate([dst, jnp.full((pad,), TRASH, dst.dtype)])
    dst2d = dst_p.reshape(NS * NB, BATCH)

    x = jnp.concatenate([user_emb, movie_emb], axis=0)
    x = jnp.pad(x, ((0, NP - N_NODES), (0, 0)))

    zeros_sc = jnp.zeros((RPT, 128), jnp.float32)
    ones_sc = jnp.ones((BATCH, 128), jnp.float32)

    deg = _deg_call(dst2d, ones_sc, zeros_sc)[0]

    h1 = _mm(x, W1, deg, 4)
    agg1 = _agg4(src_p, dst2d, h1, zeros_sc)
    x1 = _combine(agg1, h1, deg, b1.reshape(4, 1, 128), relu=True)

    h2 = _mm(x1, W2, deg, 2)
    agg2 = _agg2(src_p, dst2d, h2, zeros_sc)
    out = _combine(agg2, h2, deg, b2.reshape(2, 1, 128), relu=False)

    return out[:N_NODES]
